# Initial kernel scaffold; baseline (speedup 1.0000x reference)
import jax
import jax.numpy as jnp
from jax.experimental import pallas as pl


def kernel(points, features, mask, params):
    raise NotImplementedError("write your pallas kernel here")


# speedup vs baseline: 10.2809x; 10.2809x over previous
import functools

import jax
import jax.numpy as jnp
from jax import lax
from jax.experimental import pallas as pl

EPSV = 1e-5
KNB = 7


def _accum_stats(st_ref, st):
    @pl.when(pl.program_id(0) == 0)
    def _():
        st_ref[...] = jnp.zeros_like(st_ref)

    st_ref[...] += st


def _fstats_kernel(f_ref, m_ref, st_ref):
    t = f_ref[...] * m_ref[...]
    s = jnp.sum(t, axis=(0, 2))
    q = jnp.sum(t * t, axis=(0, 2))
    _accum_stats(st_ref, jnp.concatenate([s[None], q[None]], axis=0))


def _fstats(fts, msk, bb):
    B, cf, n = fts.shape
    return pl.pallas_call(
        _fstats_kernel,
        grid=(B // bb,),
        in_specs=[
            pl.BlockSpec((bb, cf, n), lambda i: (i, 0, 0)),
            pl.BlockSpec((bb, 1, n), lambda i: (i, 0, 0)),
        ],
        out_specs=pl.BlockSpec((2, cf), lambda i: (0, 0)),
        out_shape=jax.ShapeDtypeStruct((2, cf), jnp.float32),
    )(fts, msk)


def _ec_front_kernel(pts_ref, fts_ref, msk_ref, au_ref, aw_ref, asc_ref,
                     bu_ref, bw_ref, bsc_ref, pre0_ref, presc_ref, st_ref):
    m = msk_ref[...]
    n = m.shape[2]
    t = fts_ref[...] * m
    shift = jnp.where(m == 0.0, 1e9, 0.0)
    pm = pts_ref[...] * m + shift
    xx = jnp.sum(pm * pm, axis=1)
    inner = lax.dot_general(pm, pm, (((1,), (1,)), ((0,), (0,))),
                            preferred_element_type=jnp.float32)
    negd = -((xx[:, :, None] - 2.0 * inner) + xx[:, None, :])

    iota_m = lax.broadcasted_iota(jnp.int32, negd.shape, 2)
    idxs = []
    nd = negd
    for j in range(KNB + 1):
        mval = jnp.max(nd, axis=2, keepdims=True)
        first = jnp.min(jnp.where(nd == mval, iota_m, n), axis=2)
        if j:
            idxs.append(first)
        nd = jnp.where(iota_m == first[:, :, None], -jnp.inf, nd)

    u = jnp.einsum('oc,bcn->bon', au_ref[...], t,
                   preferred_element_type=jnp.float32) + bu_ref[...][None] * m
    w = jnp.einsum('oc,bcn->bon', aw_ref[...], t,
                   preferred_element_type=jnp.float32) + bw_ref[...][None] * m
    presc = jnp.einsum('oc,bcn->bon', asc_ref[...], t,
                       preferred_element_type=jnp.float32) + bsc_ref[...][None] * m

    idx_flat = jnp.concatenate(idxs, axis=1)
    onehot = (lax.broadcasted_iota(jnp.int32, (m.shape[0], n, KNB * n), 1)
              == idx_flat[:, None, :]).astype(jnp.float32)
    nbr = lax.dot_general(w, onehot, (((2,), (1,)), ((0,), (0,))),
                          preferred_element_type=jnp.float32)
    u_exp = jnp.concatenate([u] * KNB, axis=2)
    pre0 = nbr + u_exp

    pre0_ref[...] = pre0
    presc_ref[...] = presc
    st = jnp.concatenate([
        jnp.sum(pre0, axis=(0, 2))[None],
        jnp.sum(pre0 * pre0, axis=(0, 2))[None],
        jnp.sum(presc, axis=(0, 2))[None],
        jnp.sum(presc * presc, axis=(0, 2))[None],
    ], axis=0)
    _accum_stats(st_ref, st)


def _ec_front(pts, fts, msk, au, aw, asc, bu, bw, bsc, bb):
    B, cp, n = pts.shape
    cf = fts.shape[1]
    co = au.shape[0]
    E = KNB * n
    return pl.pallas_call(
        _ec_front_kernel,
        grid=(B // bb,),
        in_specs=[
            pl.BlockSpec((bb, cp, n), lambda i: (i, 0, 0)),
            pl.BlockSpec((bb, cf, n), lambda i: (i, 0, 0)),
            pl.BlockSpec((bb, 1, n), lambda i: (i, 0, 0)),
            pl.BlockSpec((co, cf), lambda i: (0, 0)),
            pl.BlockSpec((co, cf), lambda i: (0, 0)),
            pl.BlockSpec((co, cf), lambda i: (0, 0)),
            pl.BlockSpec((co, 1), lambda i: (0, 0)),
            pl.BlockSpec((co, 1), lambda i: (0, 0)),
            pl.BlockSpec((co, 1), lambda i: (0, 0)),
        ],
        out_specs=[
            pl.BlockSpec((bb, co, E), lambda i: (i, 0, 0)),
            pl.BlockSpec((bb, co, n), lambda i: (i, 0, 0)),
            pl.BlockSpec((4, co), lambda i: (0, 0)),
        ],
        out_shape=[
            jax.ShapeDtypeStruct((B, co, E), jnp.float32),
            jax.ShapeDtypeStruct((B, co, n), jnp.float32),
            jax.ShapeDtypeStruct((4, co), jnp.float32),
        ],
    )(pts, fts, msk, au, aw, asc, bu, bw, bsc)


def _layer_kernel(pre_ref, al_ref, be_ref, w_ref, out_ref, st_ref):
    a = jnp.maximum(al_ref[...][None] * pre_ref[...] + be_ref[...][None], 0.0)
    o = jnp.einsum('oc,bce->boe', w_ref[...], a,
                   preferred_element_type=jnp.float32)
    out_ref[...] = o
    st = jnp.concatenate([
        jnp.sum(o, axis=(0, 2))[None],
        jnp.sum(o * o, axis=(0, 2))[None],
    ], axis=0)
    _accum_stats(st_ref, st)


def _layer(pre, al, be, w, bb):
    B, ci, E = pre.shape
    co = w.shape[0]
    return pl.pallas_call(
        _layer_kernel,
        grid=(B // bb,),
        in_specs=[
            pl.BlockSpec((bb, ci, E), lambda i: (i, 0, 0)),
            pl.BlockSpec((ci, 1), lambda i: (0, 0)),
            pl.BlockSpec((ci, 1), lambda i: (0, 0)),
            pl.BlockSpec((co, ci), lambda i: (0, 0)),
        ],
        out_specs=[
            pl.BlockSpec((bb, co, E), lambda i: (i, 0, 0)),
            pl.BlockSpec((2, co), lambda i: (0, 0)),
        ],
        out_shape=[
            jax.ShapeDtypeStruct((B, co, E), jnp.float32),
            jax.ShapeDtypeStruct((2, co), jnp.float32),
        ],
    )(pre, al, be, w)


def _ecout_kernel(pre2_ref, presc_ref, msk_ref, a2_ref, b2_ref, asc_ref,
                  bsc_ref, out_ref):
    n = presc_ref.shape[2]
    a2 = jnp.maximum(a2_ref[...][None] * pre2_ref[...] + b2_ref[...][None], 0.0)
    f = a2[:, :, 0:n]
    for j in range(1, KNB):
        f = f + a2[:, :, j * n:(j + 1) * n]
    f = f * (1.0 / KNB)
    sc = asc_ref[...][None] * presc_ref[...] + bsc_ref[...][None]
    out_ref[...] = jnp.maximum(sc + f, 0.0) * msk_ref[...]


def _ecout(pre2, presc, msk, a2, b2, asc, bsc, bb):
    B, co, E = pre2.shape
    n = presc.shape[2]
    return pl.pallas_call(
        _ecout_kernel,
        grid=(B // bb,),
        in_specs=[
            pl.BlockSpec((bb, co, E), lambda i: (i, 0, 0)),
            pl.BlockSpec((bb, co, n), lambda i: (i, 0, 0)),
            pl.BlockSpec((bb, 1, n), lambda i: (i, 0, 0)),
            pl.BlockSpec((co, 1), lambda i: (0, 0)),
            pl.BlockSpec((co, 1), lambda i: (0, 0)),
            pl.BlockSpec((co, 1), lambda i: (0, 0)),
            pl.BlockSpec((co, 1), lambda i: (0, 0)),
        ],
        out_specs=pl.BlockSpec((bb, co, n), lambda i: (i, 0, 0)),
        out_shape=jax.ShapeDtypeStruct((B, co, n), jnp.float32),
    )(pre2, presc, msk, a2, b2, asc, bsc)


def _fuse_kernel(o0_ref, o1_ref, w0_ref, w1_ref, out_ref, st_ref):
    pre = (jnp.einsum('oc,bcn->bon', w0_ref[...], o0_ref[...],
                      preferred_element_type=jnp.float32)
           + jnp.einsum('oc,bcn->bon', w1_ref[...], o1_ref[...],
                        preferred_element_type=jnp.float32))
    out_ref[...] = pre
    st = jnp.concatenate([
        jnp.sum(pre, axis=(0, 2))[None],
        jnp.sum(pre * pre, axis=(0, 2))[None],
    ], axis=0)
    _accum_stats(st_ref, st)


def _fuse(o0, o1, w0, w1, bb):
    B, c0, n = o0.shape
    c1 = o1.shape[1]
    co = w0.shape[0]
    return pl.pallas_call(
        _fuse_kernel,
        grid=(B // bb,),
        in_specs=[
            pl.BlockSpec((bb, c0, n), lambda i: (i, 0, 0)),
            pl.BlockSpec((bb, c1, n), lambda i: (i, 0, 0)),
            pl.BlockSpec((co, c0), lambda i: (0, 0)),
            pl.BlockSpec((co, c1), lambda i: (0, 0)),
        ],
        out_specs=[
            pl.BlockSpec((bb, co, n), lambda i: (i, 0, 0)),
            pl.BlockSpec((2, co), lambda i: (0, 0)),
        ],
        out_shape=[
            jax.ShapeDtypeStruct((B, co, n), jnp.float32),
            jax.ShapeDtypeStruct((2, co), jnp.float32),
        ],
    )(o0, o1, w0, w1)


def _head_kernel(pre_ref, msk_ref, afu_ref, bfu_ref, wfc1_ref, bfc1_ref,
                 wout_ref, bout_ref, out_ref):
    m = msk_ref[...]
    x = jnp.maximum(afu_ref[...][None] * pre_ref[...] + bfu_ref[...][None],
                    0.0) * m
    cnt = jnp.maximum(jnp.sum(m, axis=2), 1.0)
    h = jnp.sum(x, axis=2) / cnt
    h1 = lax.dot_general(h, wfc1_ref[...], (((1,), (1,)), ((), ())),
                         preferred_element_type=jnp.float32) + bfc1_ref[...]
    h1 = jnp.maximum(h1, 0.0)
    out = lax.dot_general(h1, wout_ref[...], (((1,), (1,)), ((), ())),
                          preferred_element_type=jnp.float32) + bout_ref[...]
    out_ref[...] = out


def _head(pre, msk, afu, bfu, wfc1, bfc1, wout, bout, bb):
    B, cfu, n = pre.shape
    ch = wfc1.shape[0]
    co = wout.shape[0]
    return pl.pallas_call(
        _head_kernel,
        grid=(B // bb,),
        in_specs=[
            pl.BlockSpec((bb, cfu, n), lambda i: (i, 0, 0)),
            pl.BlockSpec((bb, 1, n), lambda i: (i, 0, 0)),
            pl.BlockSpec((cfu, 1), lambda i: (0, 0)),
            pl.BlockSpec((cfu, 1), lambda i: (0, 0)),
            pl.BlockSpec((ch, cfu), lambda i: (0, 0)),
            pl.BlockSpec((1, ch), lambda i: (0, 0)),
            pl.BlockSpec((co, ch), lambda i: (0, 0)),
            pl.BlockSpec((1, co), lambda i: (0, 0)),
        ],
        out_specs=pl.BlockSpec((bb, co), lambda i: (i, 0)),
        out_shape=jax.ShapeDtypeStruct((B, co), jnp.float32),
    )(pre, msk, afu, bfu, wfc1, bfc1, wout, bout)


def _affine_from_stats(s, q, cnt, g, b):
    mean = s / cnt
    var = q / cnt - mean * mean
    al = g / jnp.sqrt(var + EPSV)
    be = b - mean * al
    return al, be


def _edge_conv_block(pts, fts, msk, p, nedges, nnodes, au, aw, bu, bw,
                     bb_front, bb_layer):
    asc = p['_asc']
    bsc = p['_bsc']
    pre0, presc, st = _ec_front(pts, fts, msk, au, aw, asc, bu, bw, bsc,
                                bb_front)
    al0, be0 = _affine_from_stats(st[0], st[1], nedges, p['g0'], p['b0'])
    alsc, besc = _affine_from_stats(st[2], st[3], nnodes, p['gsc'], p['bsc'])
    pre1, st1 = _layer(pre0, al0[:, None], be0[:, None], p['W1'], bb_layer)
    al1, be1 = _affine_from_stats(st1[0], st1[1], nedges, p['g1'], p['b1'])
    pre2, st2 = _layer(pre1, al1[:, None], be1[:, None], p['W2'], bb_layer)
    al2, be2 = _affine_from_stats(st2[0], st2[1], nedges, p['g2'], p['b2'])
    return _ecout(pre2, presc, msk, al2[:, None], be2[:, None],
                  alsc[:, None], besc[:, None], bb_front)


def kernel(points, features, mask, params):
    B, cf, n = features.shape
    nnodes = B * n
    nedges = B * n * KNB
    bb_front = 4
    bb_layer = 8

    stf = _fstats(features, mask, 8)
    alf, bef = _affine_from_stats(stf[0], stf[1], nnodes,
                                  params['bn_fts_g'], params['bn_fts_b'])

    p0 = dict(params['ec0'])
    w0a, w0b = p0['W0'][:, :cf], p0['W0'][:, cf:]
    au0 = (w0a - w0b) * alf[None, :]
    bu0 = ((w0a - w0b) @ bef)[:, None]
    aw0 = w0b * alf[None, :]
    bw0 = (w0b @ bef)[:, None]
    p0['_asc'] = p0['Wsc'] * alf[None, :]
    p0['_bsc'] = (p0['Wsc'] @ bef)[:, None]
    out0 = _edge_conv_block(points, features, mask, p0, nedges, nnodes,
                            au0, aw0, bu0, bw0, bb_front, bb_layer)

    p1 = dict(params['ec1'])
    c1 = out0.shape[1]
    w0a1, w0b1 = p1['W0'][:, :c1], p1['W0'][:, c1:]
    zb = jnp.zeros((p1['W0'].shape[0], 1), jnp.float32)
    p1['_asc'] = p1['Wsc']
    p1['_bsc'] = zb
    out1 = _edge_conv_block(out0, out0, mask, p1, nedges, nnodes,
                            w0a1 - w0b1, w0b1, zb, zb, bb_front, bb_layer)

    wfu = params['Wfu']
    prefu, stfu = _fuse(out0, out1, wfu[:, :c1], wfu[:, c1:], bb_layer)
    alfu, befu = _affine_from_stats(stfu[0], stfu[1], nnodes,
                                    params['gfu'], params['bfu'])
    return _head(prefu, mask, alfu[:, None], befu[:, None],
                 params['Wfc1'], params['bfc1'][None, :],
                 params['Wout'], params['bout'][None, :], 8)


# speedup vs baseline: 14.5563x; 1.4159x over previous
import functools

import jax
import jax.numpy as jnp
from jax import lax
from jax.experimental import pallas as pl
from jax.experimental.pallas import tpu as pltpu
from jax.experimental.pallas import tpu_sc as plsc

EPSV = 1e-5
KNB = 7


def _accum_stats(st_ref, st):
    @pl.when(pl.program_id(0) == 0)
    def _():
        st_ref[...] = jnp.zeros_like(st_ref)

    st_ref[...] += st


def _fstats_kernel(f_ref, m_ref, st_ref):
    t = f_ref[...] * m_ref[...]
    s = jnp.sum(t, axis=(0, 2))
    q = jnp.sum(t * t, axis=(0, 2))
    _accum_stats(st_ref, jnp.concatenate([s[None], q[None]], axis=0))


def _fstats(fts, msk, bb):
    B, cf, n = fts.shape
    return pl.pallas_call(
        _fstats_kernel,
        grid=(B // bb,),
        in_specs=[
            pl.BlockSpec((bb, cf, n), lambda i: (i, 0, 0)),
            pl.BlockSpec((bb, 1, n), lambda i: (i, 0, 0)),
        ],
        out_specs=pl.BlockSpec((2, cf), lambda i: (0, 0)),
        out_shape=jax.ShapeDtypeStruct((2, cf), jnp.float32),
    )(fts, msk)


def _ec_front_kernel(pts_ref, fts_ref, msk_ref, au_ref, aw_ref, asc_ref,
                     bu_ref, bw_ref, bsc_ref, pre0_ref, presc_ref, st_ref):
    m = msk_ref[...]
    n = m.shape[2]
    t = fts_ref[...] * m
    pm = pts_ref[...] * m + jnp.where(m == 0.0, 1e9, 0.0)
    xx = jnp.sum(pm * pm, axis=1)
    inner = lax.dot_general(pm, pm, (((1,), (1,)), ((0,), (0,))),
                            preferred_element_type=jnp.float32)
    negd = -((xx[:, None, :] - 2.0 * inner) + xx[:, :, None])

    iota_m = lax.broadcasted_iota(jnp.int32, negd.shape, 1)
    sels = []
    nd = negd
    for j in range(KNB + 1):
        mval = jnp.max(nd, axis=1, keepdims=True)
        first = jnp.min(jnp.where(nd == mval, iota_m, n), axis=1)
        sel = iota_m == first[:, None, :]
        if j:
            sels.append(sel.astype(jnp.float32))
        nd = jnp.where(sel, -jnp.inf, nd)

    u = jnp.einsum('oc,bcn->bon', au_ref[...], t,
                   preferred_element_type=jnp.float32) + bu_ref[...][None] * m
    w = jnp.einsum('oc,bcn->bon', aw_ref[...], t,
                   preferred_element_type=jnp.float32) + bw_ref[...][None] * m
    presc = jnp.einsum('oc,bcn->bon', asc_ref[...], t,
                       preferred_element_type=jnp.float32) + bsc_ref[...][None] * m

    pre0 = jnp.concatenate(
        [lax.dot_general(w, sj, (((2,), (1,)), ((0,), (0,))),
                         preferred_element_type=jnp.float32) + u
         for sj in sels], axis=2)

    pre0_ref[...] = pre0
    presc_ref[...] = presc
    st = jnp.concatenate([
        jnp.sum(pre0, axis=(0, 2))[None],
        jnp.sum(pre0 * pre0, axis=(0, 2))[None],
        jnp.sum(presc, axis=(0, 2))[None],
        jnp.sum(presc * presc, axis=(0, 2))[None],
    ], axis=0)
    _accum_stats(st_ref, st)


def _ec_front(pts, fts, msk, au, aw, asc, bu, bw, bsc, bb):
    B, cp, n = pts.shape
    cf = fts.shape[1]
    co = au.shape[0]
    E = KNB * n
    return pl.pallas_call(
        _ec_front_kernel,
        grid=(B // bb,),
        in_specs=[
            pl.BlockSpec((bb, cp, n), lambda i: (i, 0, 0)),
            pl.BlockSpec((bb, cf, n), lambda i: (i, 0, 0)),
            pl.BlockSpec((bb, 1, n), lambda i: (i, 0, 0)),
            pl.BlockSpec((co, cf), lambda i: (0, 0)),
            pl.BlockSpec((co, cf), lambda i: (0, 0)),
            pl.BlockSpec((co, cf), lambda i: (0, 0)),
            pl.BlockSpec((co, 1), lambda i: (0, 0)),
            pl.BlockSpec((co, 1), lambda i: (0, 0)),
            pl.BlockSpec((co, 1), lambda i: (0, 0)),
        ],
        out_specs=[
            pl.BlockSpec((bb, co, E), lambda i: (i, 0, 0)),
            pl.BlockSpec((bb, co, n), lambda i: (i, 0, 0)),
            pl.BlockSpec((4, co), lambda i: (0, 0)),
        ],
        out_shape=[
            jax.ShapeDtypeStruct((B, co, E), jnp.float32),
            jax.ShapeDtypeStruct((B, co, n), jnp.float32),
            jax.ShapeDtypeStruct((4, co), jnp.float32),
        ],
    )(pts, fts, msk, au, aw, asc, bu, bw, bsc)


def _layer_kernel(pre_ref, al_ref, be_ref, w_ref, out_ref, st_ref):
    w = w_ref[...]
    al, be = al_ref[...], be_ref[...]
    s = q = 0.0
    for i in range(pre_ref.shape[0]):
        a = jnp.maximum(al * pre_ref[i] + be, 0.0)
        o = jnp.dot(w, a, preferred_element_type=jnp.float32)
        out_ref[i] = o
        s = s + jnp.sum(o, axis=1)
        q = q + jnp.sum(o * o, axis=1)
    _accum_stats(st_ref, jnp.concatenate([s[None], q[None]], axis=0))


def _layer(pre, al, be, w, bb):
    B, ci, E = pre.shape
    co = w.shape[0]
    return pl.pallas_call(
        _layer_kernel,
        grid=(B // bb,),
        in_specs=[
            pl.BlockSpec((bb, ci, E), lambda i: (i, 0, 0)),
            pl.BlockSpec((ci, 1), lambda i: (0, 0)),
            pl.BlockSpec((ci, 1), lambda i: (0, 0)),
            pl.BlockSpec((co, ci), lambda i: (0, 0)),
        ],
        out_specs=[
            pl.BlockSpec((bb, co, E), lambda i: (i, 0, 0)),
            pl.BlockSpec((2, co), lambda i: (0, 0)),
        ],
        out_shape=[
            jax.ShapeDtypeStruct((B, co, E), jnp.float32),
            jax.ShapeDtypeStruct((2, co), jnp.float32),
        ],
    )(pre, al, be, w)


def _ecout_kernel(pre2_ref, presc_ref, msk_ref, a2_ref, b2_ref, asc_ref,
                  bsc_ref, out_ref):
    n = presc_ref.shape[2]
    a2 = jnp.maximum(a2_ref[...][None] * pre2_ref[...] + b2_ref[...][None], 0.0)
    f = a2[:, :, 0:n]
    for j in range(1, KNB):
        f = f + a2[:, :, j * n:(j + 1) * n]
    f = f * (1.0 / KNB)
    sc = asc_ref[...][None] * presc_ref[...] + bsc_ref[...][None]
    out_ref[...] = jnp.maximum(sc + f, 0.0) * msk_ref[...]


def _ecout(pre2, presc, msk, a2, b2, asc, bsc, bb):
    B, co, E = pre2.shape
    n = presc.shape[2]
    return pl.pallas_call(
        _ecout_kernel,
        grid=(B // bb,),
        in_specs=[
            pl.BlockSpec((bb, co, E), lambda i: (i, 0, 0)),
            pl.BlockSpec((bb, co, n), lambda i: (i, 0, 0)),
            pl.BlockSpec((bb, 1, n), lambda i: (i, 0, 0)),
            pl.BlockSpec((co, 1), lambda i: (0, 0)),
            pl.BlockSpec((co, 1), lambda i: (0, 0)),
            pl.BlockSpec((co, 1), lambda i: (0, 0)),
            pl.BlockSpec((co, 1), lambda i: (0, 0)),
        ],
        out_specs=pl.BlockSpec((bb, co, n), lambda i: (i, 0, 0)),
        out_shape=jax.ShapeDtypeStruct((B, co, n), jnp.float32),
    )(pre2, presc, msk, a2, b2, asc, bsc)


def _ec_front_sc_kernel(pts_ref, fts_ref, msk_ref, au_ref, aw_ref, asc_ref,
                        bu_ref, bw_ref, bsc_ref,
                        wt_ref, u_ref, idx_ref, presc_ref, st_ref):
    m = msk_ref[...]
    bb, _, n = m.shape
    t = fts_ref[...] * m
    pm = pts_ref[...] * m + jnp.where(m == 0.0, 1e9, 0.0)
    xx = jnp.sum(pm * pm, axis=1)
    inner = lax.dot_general(pm, pm, (((1,), (1,)), ((0,), (0,))),
                            preferred_element_type=jnp.float32)
    negd = -((xx[:, None, :] - 2.0 * inner) + xx[:, :, None])

    iota_m = lax.broadcasted_iota(jnp.int32, negd.shape, 1)
    firsts = []
    adj = 0.0
    nd = negd
    for j in range(KNB + 1):
        mval = jnp.max(nd, axis=1, keepdims=True)
        first = jnp.min(jnp.where(nd == mval, iota_m, n), axis=1)
        sel = iota_m == first[:, None, :]
        if j:
            firsts.append(first)
            adj = adj + sel.astype(jnp.float32)
        nd = jnp.where(sel, -jnp.inf, nd)

    u = jnp.einsum('oc,bcn->bon', au_ref[...], t,
                   preferred_element_type=jnp.float32) + bu_ref[...][None] * m
    w = jnp.einsum('oc,bcn->bon', aw_ref[...], t,
                   preferred_element_type=jnp.float32) + bw_ref[...][None] * m
    presc = jnp.einsum('oc,bcn->bon', asc_ref[...], t,
                       preferred_element_type=jnp.float32) + bsc_ref[...][None] * m

    deg = jnp.sum(adj, axis=2)
    gsum = lax.dot_general(w, adj, (((2,), (1,)), ((0,), (0,))),
                           preferred_element_type=jnp.float32)
    s0 = (KNB * jnp.sum(u, axis=(0, 2))
          + jnp.sum(w * deg[:, None, :], axis=(0, 2)))
    q0 = (KNB * jnp.sum(u * u, axis=(0, 2))
          + 2.0 * jnp.sum(u * gsum, axis=(0, 2))
          + jnp.sum(w * w * deg[:, None, :], axis=(0, 2)))

    base = (pl.program_id(0) * bb
            + lax.broadcasted_iota(jnp.int32, (bb, 1), 0)) * n
    idx_ref[...] = jnp.concatenate([f + base for f in firsts], axis=1)
    wtr = jnp.transpose(w, (0, 2, 1))
    wt_ref[...] = jnp.concatenate([wtr, jnp.zeros_like(wtr)], axis=2)
    u_ref[...] = u
    presc_ref[...] = presc
    st = jnp.concatenate([
        s0[None], q0[None],
        jnp.sum(presc, axis=(0, 2))[None],
        jnp.sum(presc * presc, axis=(0, 2))[None],
    ], axis=0)
    _accum_stats(st_ref, st)


def _ec_front_sc(pts, fts, msk, au, aw, asc, bu, bw, bsc, bb):
    B, cp, n = pts.shape
    cf = fts.shape[1]
    co = au.shape[0]
    return pl.pallas_call(
        _ec_front_sc_kernel,
        grid=(B // bb,),
        in_specs=[
            pl.BlockSpec((bb, cp, n), lambda i: (i, 0, 0)),
            pl.BlockSpec((bb, cf, n), lambda i: (i, 0, 0)),
            pl.BlockSpec((bb, 1, n), lambda i: (i, 0, 0)),
            pl.BlockSpec((co, cf), lambda i: (0, 0)),
            pl.BlockSpec((co, cf), lambda i: (0, 0)),
            pl.BlockSpec((co, cf), lambda i: (0, 0)),
            pl.BlockSpec((co, 1), lambda i: (0, 0)),
            pl.BlockSpec((co, 1), lambda i: (0, 0)),
            pl.BlockSpec((co, 1), lambda i: (0, 0)),
        ],
        out_specs=[
            pl.BlockSpec((bb, n, 2 * co), lambda i: (i, 0, 0)),
            pl.BlockSpec((bb, co, n), lambda i: (i, 0, 0)),
            pl.BlockSpec((bb, KNB * n), lambda i: (i, 0)),
            pl.BlockSpec((bb, co, n), lambda i: (i, 0, 0)),
            pl.BlockSpec((4, co), lambda i: (0, 0)),
        ],
        out_shape=[
            jax.ShapeDtypeStruct((B, n, 2 * co), jnp.float32),
            jax.ShapeDtypeStruct((B, co, n), jnp.float32),
            jax.ShapeDtypeStruct((B, KNB * n), jnp.int32),
            jax.ShapeDtypeStruct((B, co, n), jnp.float32),
            jax.ShapeDtypeStruct((4, co), jnp.float32),
        ],
    )(pts, fts, msk, au, aw, asc, bu, bw, bsc)


def _sc_gather_rows(table, idx):
    info = plsc.get_sparse_core_info()
    nw = info.num_cores * info.num_subcores
    E, = idx.shape
    D = table.shape[1]
    e_per_w = E // nw
    ch = 448
    nch = e_per_w // ch
    mesh = plsc.VectorSubcoreMesh(core_axis_name="c", subcore_axis_name="s")

    @functools.partial(
        pl.kernel, mesh=mesh,
        out_type=jax.ShapeDtypeStruct((E, D), jnp.float32),
        scratch_types=[
            pltpu.VMEM((e_per_w,), jnp.int32),
            pltpu.VMEM((ch, D), jnp.float32),
            pltpu.VMEM((ch, D), jnp.float32),
            pltpu.SemaphoreType.DMA,
            pltpu.SemaphoreType.DMA,
            pltpu.SemaphoreType.DMA,
        ],
    )
    def k(table_hbm, idx_hbm, out_hbm, idx_v, rows0, rows1, gsem, ws0, ws1):
        wid = lax.axis_index("s") * info.num_cores + lax.axis_index("c")
        base = wid * e_per_w
        pltpu.sync_copy(idx_hbm.at[pl.ds(base, e_per_w)], idx_v)
        rows = [rows0, rows1]
        wsems = [ws0, ws1]
        wb = [None, None]
        for c in range(nch):
            b = c % 2
            if wb[b] is not None:
                wb[b].wait()
            pltpu.async_copy(table_hbm.at[idx_v.at[pl.ds(c * ch, ch)]],
                             rows[b], gsem).wait()
            wb[b] = pltpu.async_copy(
                rows[b], out_hbm.at[pl.ds(base + c * ch, ch)], wsems[b])
        for b in range(2):
            if wb[b] is not None:
                wb[b].wait()

    return k(table, idx)


def _gather_layer_kernel(g_ref, u_ref, al_ref, be_ref, w_ref, out_ref, st_ref):
    al, be, w1 = al_ref[...], be_ref[...], w_ref[...]
    s = q = 0.0
    for i in range(g_ref.shape[0]):
        gt = jnp.transpose(g_ref[i], (1, 0))[:w1.shape[1]]
        u = u_ref[i]
        pre0 = gt + jnp.concatenate([u] * KNB, axis=1)
        a = jnp.maximum(al * pre0 + be, 0.0)
        o = jnp.dot(w1, a, preferred_element_type=jnp.float32)
        out_ref[i] = o
        s = s + jnp.sum(o, axis=1)
        q = q + jnp.sum(o * o, axis=1)
    _accum_stats(st_ref, jnp.concatenate([s[None], q[None]], axis=0))


def _gather_layer(g, u, al, be, w, bb):
    B, E, gp = g.shape
    n = u.shape[2]
    ci = w.shape[1]
    co = w.shape[0]
    return pl.pallas_call(
        _gather_layer_kernel,
        grid=(B // bb,),
        in_specs=[
            pl.BlockSpec((bb, E, gp), lambda i: (i, 0, 0)),
            pl.BlockSpec((bb, ci, n), lambda i: (i, 0, 0)),
            pl.BlockSpec((ci, 1), lambda i: (0, 0)),
            pl.BlockSpec((ci, 1), lambda i: (0, 0)),
            pl.BlockSpec((co, ci), lambda i: (0, 0)),
        ],
        out_specs=[
            pl.BlockSpec((bb, co, E), lambda i: (i, 0, 0)),
            pl.BlockSpec((2, co), lambda i: (0, 0)),
        ],
        out_shape=[
            jax.ShapeDtypeStruct((B, co, E), jnp.float32),
            jax.ShapeDtypeStruct((2, co), jnp.float32),
        ],
    )(g, u, al, be, w)


def _fuse_kernel(o0_ref, o1_ref, w0_ref, w1_ref, out_ref, st_ref):
    w0, w1 = w0_ref[...], w1_ref[...]
    s = q = 0.0
    for i in range(o0_ref.shape[0]):
        pre = (jnp.dot(w0, o0_ref[i], preferred_element_type=jnp.float32)
               + jnp.dot(w1, o1_ref[i], preferred_element_type=jnp.float32))
        out_ref[i] = pre
        s = s + jnp.sum(pre, axis=1)
        q = q + jnp.sum(pre * pre, axis=1)
    _accum_stats(st_ref, jnp.concatenate([s[None], q[None]], axis=0))


def _fuse(o0, o1, w0, w1, bb):
    B, c0, n = o0.shape
    c1 = o1.shape[1]
    co = w0.shape[0]
    return pl.pallas_call(
        _fuse_kernel,
        grid=(B // bb,),
        in_specs=[
            pl.BlockSpec((bb, c0, n), lambda i: (i, 0, 0)),
            pl.BlockSpec((bb, c1, n), lambda i: (i, 0, 0)),
            pl.BlockSpec((co, c0), lambda i: (0, 0)),
            pl.BlockSpec((co, c1), lambda i: (0, 0)),
        ],
        out_specs=[
            pl.BlockSpec((bb, co, n), lambda i: (i, 0, 0)),
            pl.BlockSpec((2, co), lambda i: (0, 0)),
        ],
        out_shape=[
            jax.ShapeDtypeStruct((B, co, n), jnp.float32),
            jax.ShapeDtypeStruct((2, co), jnp.float32),
        ],
    )(o0, o1, w0, w1)


def _head_kernel(pre_ref, msk_ref, afu_ref, bfu_ref, wfc1_ref, bfc1_ref,
                 wout_ref, bout_ref, out_ref):
    m = msk_ref[...]
    x = jnp.maximum(afu_ref[...][None] * pre_ref[...] + bfu_ref[...][None],
                    0.0) * m
    cnt = jnp.maximum(jnp.sum(m, axis=2), 1.0)
    h = jnp.sum(x, axis=2) / cnt
    h1 = lax.dot_general(h, wfc1_ref[...], (((1,), (1,)), ((), ())),
                         preferred_element_type=jnp.float32) + bfc1_ref[...]
    h1 = jnp.maximum(h1, 0.0)
    out = lax.dot_general(h1, wout_ref[...], (((1,), (1,)), ((), ())),
                          preferred_element_type=jnp.float32) + bout_ref[...]
    out_ref[...] = out


def _head(pre, msk, afu, bfu, wfc1, bfc1, wout, bout, bb):
    B, cfu, n = pre.shape
    ch = wfc1.shape[0]
    co = wout.shape[0]
    return pl.pallas_call(
        _head_kernel,
        grid=(B // bb,),
        in_specs=[
            pl.BlockSpec((bb, cfu, n), lambda i: (i, 0, 0)),
            pl.BlockSpec((bb, 1, n), lambda i: (i, 0, 0)),
            pl.BlockSpec((cfu, 1), lambda i: (0, 0)),
            pl.BlockSpec((cfu, 1), lambda i: (0, 0)),
            pl.BlockSpec((ch, cfu), lambda i: (0, 0)),
            pl.BlockSpec((1, ch), lambda i: (0, 0)),
            pl.BlockSpec((co, ch), lambda i: (0, 0)),
            pl.BlockSpec((1, co), lambda i: (0, 0)),
        ],
        out_specs=pl.BlockSpec((bb, co), lambda i: (i, 0)),
        out_shape=jax.ShapeDtypeStruct((B, co), jnp.float32),
    )(pre, msk, afu, bfu, wfc1, bfc1, wout, bout)


def _affine_from_stats(s, q, cnt, g, b):
    mean = s / cnt
    var = q / cnt - mean * mean
    al = g / jnp.sqrt(var + EPSV)
    be = b - mean * al
    return al, be


def _edge_conv_block(pts, fts, msk, p, nedges, nnodes, au, aw, bu, bw,
                     bb_front, bb_layer):
    asc = p['_asc']
    bsc = p['_bsc']
    pre0, presc, st = _ec_front(pts, fts, msk, au, aw, asc, bu, bw, bsc,
                                bb_front)
    al0, be0 = _affine_from_stats(st[0], st[1], nedges, p['g0'], p['b0'])
    alsc, besc = _affine_from_stats(st[2], st[3], nnodes, p['gsc'], p['bsc'])
    pre1, st1 = _layer(pre0, al0[:, None], be0[:, None], p['W1'], bb_layer)
    al1, be1 = _affine_from_stats(st1[0], st1[1], nedges, p['g1'], p['b1'])
    pre2, st2 = _layer(pre1, al1[:, None], be1[:, None], p['W2'], bb_layer)
    al2, be2 = _affine_from_stats(st2[0], st2[1], nedges, p['g2'], p['b2'])
    return _ecout(pre2, presc, msk, al2[:, None], be2[:, None],
                  alsc[:, None], besc[:, None], bb_front)


def kernel(points, features, mask, params):
    B, cf, n = features.shape
    nnodes = B * n
    nedges = B * n * KNB
    bb_front = 4
    bb_layer = 8

    stf = _fstats(features, mask, 8)
    alf, bef = _affine_from_stats(stf[0], stf[1], nnodes,
                                  params['bn_fts_g'], params['bn_fts_b'])

    p0 = dict(params['ec0'])
    w0a, w0b = p0['W0'][:, :cf], p0['W0'][:, cf:]
    au0 = (w0a - w0b) * alf[None, :]
    bu0 = ((w0a - w0b) @ bef)[:, None]
    aw0 = w0b * alf[None, :]
    bw0 = (w0b @ bef)[:, None]
    p0['_asc'] = p0['Wsc'] * alf[None, :]
    p0['_bsc'] = (p0['Wsc'] @ bef)[:, None]
    out0 = _edge_conv_block(points, features, mask, p0, nedges, nnodes,
                            au0, aw0, bu0, bw0, bb_front, bb_layer)

    p1 = params['ec1']
    c1 = out0.shape[1]
    co1 = p1['W0'].shape[0]
    w0a1, w0b1 = p1['W0'][:, :c1], p1['W0'][:, c1:]
    zb = jnp.zeros((co1, 1), jnp.float32)
    nh = 2
    bh = B // nh
    fr = [_ec_front_sc(out0[h * bh:(h + 1) * bh], out0[h * bh:(h + 1) * bh],
                       mask[h * bh:(h + 1) * bh], w0a1 - w0b1, w0b1,
                       p1['Wsc'], zb, zb, zb, 8) for h in range(nh)]
    stf1 = fr[0][4] + fr[1][4]
    al0, be0 = _affine_from_stats(stf1[0], stf1[1], nedges, p1['g0'], p1['b0'])
    alsc, besc = _affine_from_stats(stf1[2], stf1[3], nnodes,
                                    p1['gsc'], p1['bsc'])

    g3 = [_sc_gather_rows(fr[h][0].reshape(bh * n, 2 * co1),
                          fr[h][2].reshape(-1)).reshape(bh, KNB * n, 2 * co1)
          for h in range(nh)]
    lay1 = [_gather_layer(g3[h], fr[h][1], al0[:, None],
                          be0[:, None], p1['W1'], bb_layer) for h in range(nh)]
    st1 = lay1[0][1] + lay1[1][1]
    al1, be1 = _affine_from_stats(st1[0], st1[1], nedges, p1['g1'], p1['b1'])
    lay2 = [_layer(lay1[h][0], al1[:, None], be1[:, None], p1['W2'], bb_layer)
            for h in range(nh)]
    st2 = lay2[0][1] + lay2[1][1]
    al2, be2 = _affine_from_stats(st2[0], st2[1], nedges, p1['g2'], p1['b2'])
    out1_h = [_ecout(lay2[h][0], fr[h][3],
                     mask[h * bh:(h + 1) * bh], al2[:, None], be2[:, None],
                     alsc[:, None], besc[:, None], bb_front)
              for h in range(nh)]

    wfu = params['Wfu']
    fu = [_fuse(out0[h * bh:(h + 1) * bh], out1_h[h], wfu[:, :c1], wfu[:, c1:],
                bb_layer) for h in range(nh)]
    stfu = fu[0][1] + fu[1][1]
    alfu, befu = _affine_from_stats(stfu[0], stfu[1], nnodes,
                                    params['gfu'], params['bfu'])
    outs = [_head(fu[h][0], mask[h * bh:(h + 1) * bh], alfu[:, None],
                  befu[:, None], params['Wfc1'], params['bfc1'][None, :],
                  params['Wout'], params['bout'][None, :], 8) for h in range(nh)]
    return jnp.concatenate(outs, axis=0)
